# baseline (speedup 1.0000x reference)
import functools

import jax
import jax.numpy as jnp
from jax import lax
from jax.experimental import pallas as pl
from jax.experimental.pallas import tpu as pltpu
from jax.experimental.pallas import tpu_sc as plsc

_ALPHA = 0.1
_BETA = 0.5

_NC = 2
_NS = 16
_NW = _NC * _NS
_C = 80


_NBUF = 4
_NMETA = 8


def _spmm_body(n_pad, n_groups, lanes,
               x_hbm, meta_hbm, w_hbm, zero_hbm, out_hbm,
               meta_bufs, w_bufs, rows_bufs, msems, wsems, gsems, ssems,
               acc_sh):
  cid = lax.axis_index("c")
  sid = lax.axis_index("s")
  wid = cid * _NS + sid
  stripe = n_pad // _NS

  pltpu.sync_copy(zero_hbm.at[pl.ds(sid * stripe, stripe)],
                  acc_sh.at[pl.ds(sid * stripe, stripe)])
  plsc.subcore_barrier()

  d = rows_bufs[0].shape[1]

  def meta_fetch(g, m):
    pltpu.async_copy(meta_hbm.at[wid, g], meta_bufs[m], msems[m])
    pltpu.async_copy(w_hbm.at[wid, g], w_bufs[m], wsems[m])

  def wait_meta(g, m):
    pltpu.make_async_copy(meta_hbm.at[wid, g], meta_bufs[m],
                          msems[m]).wait()
    pltpu.make_async_copy(w_hbm.at[wid, g], w_bufs[m], wsems[m]).wait()

  def gather(m, b):
    pltpu.async_copy(x_hbm.at[meta_bufs[m].at[0]], rows_bufs[b], gsems[b])

  def wait_gather(m, b):
    pltpu.make_async_copy(x_hbm.at[meta_bufs[m].at[0]], rows_bufs[b],
                          gsems[b]).wait()

  def scatter(m, b):
    pltpu.async_copy(rows_bufs[b], acc_sh.at[meta_bufs[m].at[1]],
                     ssems[b], add=True)

  def wait_scatter(m, b):
    pltpu.make_async_copy(rows_bufs[b], acc_sh.at[meta_bufs[m].at[1]],
                          ssems[b]).wait()

  def scale(m, b):
    rows_v = rows_bufs[b]
    w_v = w_bufs[m]

    def subblock(sb, carry):
      wv = w_v[pl.ds(sb * lanes, lanes)]
      for i in range(lanes):
        e_row = sb * lanes + i
        w = wv[i]
        for j in range(d // lanes):
          sl = pl.ds(j * lanes, lanes)
          rows_v[e_row, sl] = rows_v[e_row, sl] * w
      return carry

    lax.fori_loop(0, _C // lanes, subblock, 0)

  for g0 in range(_NBUF):
    meta_fetch(g0, g0)
  wait_meta(0, 0)
  gather(0, 0)
  wait_meta(1, 1)
  gather(1, 1)

  def group(p, carry):
    for u in range(_NMETA):
      g = p * _NMETA + u
      b = u % _NBUF
      wait_gather(u, b)
      scale(u, b)
      scatter(u, b)
      pb = (b + 2) % _NBUF
      pm = (u + 2) % _NMETA

      @pl.when(g >= 2)
      def _():
        wait_scatter(pm, pb)

      @pl.when(g + 4 < n_groups)
      def _():
        meta_fetch(g + 4, (u + 4) % _NMETA)

      @pl.when(g + 2 < n_groups)
      def _():
        wait_meta(g + 2, pm)
        gather(pm, pb)
    return carry

  lax.fori_loop(0, n_groups // _NMETA, group, 0)
  wait_scatter((n_groups - 2) % _NMETA, (n_groups - 2) % _NBUF)
  wait_scatter((n_groups - 1) % _NMETA, (n_groups - 1) % _NBUF)
  plsc.subcore_barrier()

  pltpu.sync_copy(acc_sh.at[pl.ds(sid * stripe, stripe)],
                  out_hbm.at[cid, pl.ds(sid * stripe, stripe)])


def _dense_body(p0_ref, p1_ref, ix_ref, wt_ref, b_ref, o_ref):
  hid = (1.0 - _ALPHA) * (p0_ref[...] + p1_ref[...]) + _ALPHA * ix_ref[...]
  lin = jnp.dot(hid, wt_ref[...], preferred_element_type=jnp.float32)
  o_ref[...] = _BETA * (lin + b_ref[...]) + (1.0 - _BETA) * hid


def kernel(x, init_x, edge_index, edge_weight, W, b):
  n, d = x.shape
  e = edge_weight.shape[0]
  n_groups = -(-(-(-e // (_NW * _C))) // _NMETA) * _NMETA
  e_pad = _NW * n_groups * _C

  src = edge_index[0]
  dst = edge_index[1]
  ew = edge_weight
  if e_pad != e:
    pad = e_pad - e
    src = jnp.concatenate([src, jnp.zeros((pad,), src.dtype)])
    dst = jnp.concatenate([dst, jnp.zeros((pad,), dst.dtype)])
    ew = jnp.concatenate([ew, jnp.zeros((pad,), ew.dtype)])
  meta = jnp.stack([src.reshape(_NW, n_groups, _C),
                    dst.reshape(_NW, n_groups, _C)], axis=2)
  wgrp = ew.reshape(_NW, n_groups, _C)
  n_pad = -(-n // 128) * 128
  zero_nd = jnp.zeros((n_pad, d), x.dtype)

  info = plsc.get_sparse_core_info()
  lanes = info.num_lanes
  mesh = plsc.VectorSubcoreMesh(core_axis_name="c", subcore_axis_name="s")
  spmm = pl.kernel(
      functools.partial(_spmm_body, n_pad, n_groups, lanes),
      out_type=jax.ShapeDtypeStruct((_NC, n_pad, d), jnp.float32),
      mesh=mesh,
      scratch_types=[
          [pltpu.VMEM((2, _C), jnp.int32) for _ in range(_NMETA)],
          [pltpu.VMEM((_C,), jnp.float32) for _ in range(_NMETA)],
          [pltpu.VMEM((_C, d), jnp.float32) for _ in range(_NBUF)],
          [pltpu.SemaphoreType.DMA for _ in range(_NMETA)],
          [pltpu.SemaphoreType.DMA for _ in range(_NMETA)],
          [pltpu.SemaphoreType.DMA for _ in range(_NBUF)],
          [pltpu.SemaphoreType.DMA for _ in range(_NBUF)],
          pltpu.VMEM_SHARED((n_pad, d), jnp.float32),
      ],
  )
  partial = spmm(x, meta, wgrp, zero_nd)

  bn = 1000
  wt = W.T
  b2 = b.reshape(1, d)
  return pl.pallas_call(
      _dense_body,
      grid=(n // bn,),
      in_specs=[
          pl.BlockSpec((bn, d), lambda i: (i, 0)),
          pl.BlockSpec((bn, d), lambda i: (i, 0)),
          pl.BlockSpec((bn, d), lambda i: (i, 0)),
          pl.BlockSpec((d, d), lambda i: (0, 0)),
          pl.BlockSpec((1, d), lambda i: (0, 0)),
      ],
      out_specs=pl.BlockSpec((bn, d), lambda i: (i, 0)),
      out_shape=jax.ShapeDtypeStruct((n, d), jnp.float32),
  )(partial[0, :n], partial[1, :n], init_x, wt, b2)


# speedup vs baseline: 1.0476x; 1.0476x over previous
import functools

import jax
import jax.numpy as jnp
from jax import lax
from jax.experimental import pallas as pl
from jax.experimental.pallas import tpu as pltpu
from jax.experimental.pallas import tpu_sc as plsc

_ALPHA = 0.1
_BETA = 0.5

_NC = 2
_NS = 16
_C = 80
_NBUF = 4
_NMETA = 8
_GA = 192
_GB = 64


def _spmm_body(n_pad, lanes,
               x_hbm, meta0_hbm, w0_hbm, meta1_hbm, w1_hbm, zero_hbm,
               out_hbm,
               meta_bufs, w_bufs, rows_bufs, msems, wsems, gsems, ssems,
               acc_sh):
  cid = lax.axis_index("c")
  sid = lax.axis_index("s")
  stripe = n_pad // _NS
  rsl = pl.ds(sid * stripe, stripe)
  ng = jnp.where(cid == 0, _GA, _GB)

  pltpu.sync_copy(zero_hbm.at[rsl], acc_sh.at[rsl])
  plsc.subcore_barrier()

  d = rows_bufs[0].shape[1]

  def meta_fetch(g, m):
    @pl.when(cid == 0)
    def _():
      pltpu.async_copy(meta0_hbm.at[sid, g], meta_bufs[m], msems[m])
      pltpu.async_copy(w0_hbm.at[sid, g], w_bufs[m], wsems[m])

    @pl.when(cid == 1)
    def _():
      pltpu.async_copy(meta1_hbm.at[sid, g], meta_bufs[m], msems[m])
      pltpu.async_copy(w1_hbm.at[sid, g], w_bufs[m], wsems[m])

  def wait_meta(g, m):
    pltpu.make_async_copy(meta0_hbm.at[0, 0], meta_bufs[m],
                          msems[m]).wait()
    pltpu.make_async_copy(w0_hbm.at[0, 0], w_bufs[m], wsems[m]).wait()

  def gather(m, b):
    pltpu.async_copy(x_hbm.at[meta_bufs[m].at[0]], rows_bufs[b], gsems[b])

  def wait_gather(m, b):
    pltpu.make_async_copy(x_hbm.at[meta_bufs[m].at[0]], rows_bufs[b],
                          gsems[b]).wait()

  def scatter(m, b):
    pltpu.async_copy(rows_bufs[b], acc_sh.at[meta_bufs[m].at[1]],
                     ssems[b], add=True)

  def wait_scatter(m, b):
    pltpu.make_async_copy(rows_bufs[b], acc_sh.at[meta_bufs[m].at[1]],
                          ssems[b]).wait()

  def scale(m, b):
    rows_v = rows_bufs[b]
    w_v = w_bufs[m]

    def subblock(sb, carry):
      wv = w_v[pl.ds(sb * lanes, lanes)]
      for i in range(lanes):
        e_row = sb * lanes + i
        w = wv[i]
        for j in range(d // lanes):
          sl = pl.ds(j * lanes, lanes)
          rows_v[e_row, sl] = rows_v[e_row, sl] * w
      return carry

    lax.fori_loop(0, _C // lanes, subblock, 0)

  for g0 in range(_NBUF):
    meta_fetch(g0, g0)
  wait_meta(0, 0)
  gather(0, 0)
  wait_meta(1, 1)
  gather(1, 1)

  def group(p, carry):
    for u in range(_NMETA):
      g = p * _NMETA + u
      b = u % _NBUF
      wait_gather(u, b)
      scale(u, b)
      scatter(u, b)
      pb = (b + 2) % _NBUF
      pm = (u + 2) % _NMETA

      @pl.when(g >= 2)
      def _():
        wait_scatter(pm, pb)

      @pl.when(g + 4 < ng)
      def _():
        meta_fetch(g + 4, (u + 4) % _NMETA)

      @pl.when(g + 2 < ng)
      def _():
        wait_meta(g + 2, pm)
        gather(pm, pb)
    return carry

  lax.fori_loop(0, ng // _NMETA, group, 0)
  wait_scatter(_NMETA - 2, _NBUF - 2)
  wait_scatter(_NMETA - 1, _NBUF - 1)
  plsc.subcore_barrier()

  pltpu.sync_copy(acc_sh.at[rsl], out_hbm.at[cid, rsl])


def _dense_body(p0_ref, p1_ref, ix_ref, wt_ref, b_ref, o_ref):
  hid = ((1.0 - _ALPHA) * (p0_ref[...] + p1_ref[...])
         + _ALPHA * ix_ref[...])
  lin = jnp.dot(hid, wt_ref[...], preferred_element_type=jnp.float32)
  o_ref[...] = _BETA * (lin + b_ref[...]) + (1.0 - _BETA) * hid


def kernel(x, init_x, edge_index, edge_weight, W, b):
  n, d = x.shape
  e = edge_weight.shape[0]
  e_pad = _NS * (_GA + _GB) * _C
  n0 = _NS * _GA * _C

  src = edge_index[0]
  dst = edge_index[1]
  ew = edge_weight
  if e_pad != e:
    pad = e_pad - e
    src = jnp.concatenate([src, jnp.zeros((pad,), src.dtype)])
    dst = jnp.concatenate([dst, jnp.zeros((pad,), dst.dtype)])
    ew = jnp.concatenate([ew, jnp.zeros((pad,), ew.dtype)])
  meta0 = jnp.stack([src[:n0].reshape(_NS, _GA, _C),
                     dst[:n0].reshape(_NS, _GA, _C)], axis=2)
  w0 = ew[:n0].reshape(_NS, _GA, _C)
  meta1 = jnp.stack([src[n0:].reshape(_NS, _GB, _C),
                     dst[n0:].reshape(_NS, _GB, _C)], axis=2)
  w1 = ew[n0:].reshape(_NS, _GB, _C)
  n_pad = -(-n // 128) * 128
  zero_nd = jnp.zeros((n_pad, d), x.dtype)

  info = plsc.get_sparse_core_info()
  lanes = info.num_lanes
  mesh = plsc.VectorSubcoreMesh(core_axis_name="c", subcore_axis_name="s")
  spmm = pl.kernel(
      functools.partial(_spmm_body, n_pad, lanes),
      out_type=jax.ShapeDtypeStruct((_NC, n_pad, d), jnp.float32),
      mesh=mesh,
      scratch_types=[
          [pltpu.VMEM((2, _C), jnp.int32) for _ in range(_NMETA)],
          [pltpu.VMEM((_C,), jnp.float32) for _ in range(_NMETA)],
          [pltpu.VMEM((_C, d), jnp.float32) for _ in range(_NBUF)],
          [pltpu.SemaphoreType.DMA for _ in range(_NMETA)],
          [pltpu.SemaphoreType.DMA for _ in range(_NMETA)],
          [pltpu.SemaphoreType.DMA for _ in range(_NBUF)],
          [pltpu.SemaphoreType.DMA for _ in range(_NBUF)],
          pltpu.VMEM_SHARED((n_pad, d), jnp.float32),
      ],
  )
  partial = spmm(x, meta0, w0, meta1, w1, zero_nd)

  bn = 1000
  wt = W.T
  b2 = b.reshape(1, d)
  return pl.pallas_call(
      _dense_body,
      grid=(n // bn,),
      in_specs=[
          pl.BlockSpec((bn, d), lambda i: (i, 0)),
          pl.BlockSpec((bn, d), lambda i: (i, 0)),
          pl.BlockSpec((bn, d), lambda i: (i, 0)),
          pl.BlockSpec((d, d), lambda i: (0, 0)),
          pl.BlockSpec((1, d), lambda i: (0, 0)),
      ],
      out_specs=pl.BlockSpec((bn, d), lambda i: (i, 0)),
      out_shape=jax.ShapeDtypeStruct((n, d), jnp.float32),
  )(partial[0, :n], partial[1, :n], init_x, wt, b2)
